# baseline (speedup 1.0000x reference)
import jax
import jax.numpy as jnp
from jax import lax
from jax.experimental import pallas as pl
from jax.experimental.pallas import tpu as pltpu
from jax.experimental.pallas import tpu_sc as plsc

B = 4
T = 8192
V = 128
NSC = 7
NEC = 8
EXP = 1 + NSC + NEC
ROW = V * EXP
ST = V * NSC
EN = V * NEC
BC = ST + EN
L = 16
NW = 32
TPW = T // NW
TB = 4
NCH = TPW // TB


def _tec_body(inp_hbm, enc_hbm, st_hbm, out_hbm,
              bc0, bc1, in0, in1, out0, out1, idx_v, idxi_v,
              sin0, sin1, sout0, sout1):
    wid = lax.axis_index("s") * 2 + lax.axis_index("c")
    lanes = lax.iota(jnp.int32, 16)
    bc = (bc0, bc1)
    inv = (in0, in1)
    outv = (out0, out1)
    sin = (sin0, sin1)
    sout = (sout0, sout1)

    @pl.loop(0, ST // L)
    def _(r):
        s = r * L + lanes
        v = (s * 9363) >> 16
        idx_v[pl.ds(r * L, L)] = v * 16 + (s - v * 7) + 1

    @pl.loop(0, EN // L)
    def _(r):
        e = r * L + lanes
        idx_v[pl.ds(ST + r * L, L)] = ((e >> 3) << 4) + 8 + (e & 7)

    @pl.loop(0, V // L)
    def _(r):
        v = r * L + lanes
        idxi_v[pl.ds(r * L, L)] = v << 4

    def start_in(k, t):
        pltpu.async_copy(st_hbm.at[pl.ds(t, TB)], bc[k].at[:, pl.ds(0, ST)], sin[k])
        pltpu.async_copy(enc_hbm.at[pl.ds(t, TB)], bc[k].at[:, pl.ds(ST, EN)], sin[k])
        pltpu.async_copy(inp_hbm.at[:, pl.ds(t, TB)], inv[k], sin[k])

    def wait_in(k):
        pltpu.make_async_copy(st_hbm.at[pl.ds(0, TB)], bc[k].at[:, pl.ds(0, ST)], sin[k]).wait()
        pltpu.make_async_copy(enc_hbm.at[pl.ds(0, TB)], bc[k].at[:, pl.ds(ST, EN)], sin[k]).wait()
        pltpu.make_async_copy(inp_hbm.at[:, pl.ds(0, TB)], inv[k], sin[k]).wait()

    def start_out(k, t):
        pltpu.async_copy(outv[k], out_hbm.at[:, pl.ds(t, TB)], sout[k])

    def wait_out(k):
        pltpu.make_async_copy(outv[k], out_hbm.at[:, pl.ds(0, TB)], sout[k]).wait()

    def build(k):
        for j in range(TB):
            rj = jnp.full((L,), j, jnp.int32)

            @pl.loop(0, BC // L)
            def _(r, rj=rj, j=j):
                x = bc[k][j, pl.ds(r * L, L)]
                d = idx_v[pl.ds(r * L, L)]
                for b in range(B):
                    rb = jnp.full((L,), b, jnp.int32)
                    plsc.store_scatter(outv[k], [rb, rj, d], x)

            @pl.loop(0, V // L)
            def _(r, rj=rj, j=j):
                d = idxi_v[pl.ds(r * L, L)]
                for b in range(B):
                    x = inv[k][b, j, pl.ds(r * L, L)]
                    rb = jnp.full((L,), b, jnp.int32)
                    plsc.store_scatter(outv[k], [rb, rj, d], x)

    t0 = wid * TPW
    start_in(0, t0)
    start_in(1, t0 + TB)
    for k in range(2):
        wait_in(k)
        build(k)
        start_out(k, t0 + k * TB)
        start_in(k, t0 + (k + 2) * TB)

    @pl.loop(0, (NCH - 4) // 2)
    def _(ii):
        c = 2 + ii * 2
        for k in range(2):
            t = t0 + (c + k) * TB
            wait_in(k)
            wait_out(k)
            build(k)
            start_out(k, t)
            start_in(k, t + 2 * TB)

    for k in range(2):
        wait_in(k)
        wait_out(k)
        build(k)
        start_out(k, t0 + (NCH - 2 + k) * TB)
    for k in range(2):
        wait_out(k)


@jax.jit
def kernel(inp, variable_encodings, static_channels):
    run = pl.kernel(
        _tec_body,
        out_type=jax.ShapeDtypeStruct((B, T, ROW), jnp.float32),
        mesh=plsc.VectorSubcoreMesh(core_axis_name="c", subcore_axis_name="s"),
        compiler_params=pltpu.CompilerParams(
            needs_layout_passes=False, disable_bounds_checks=True),
        scratch_types=[
            pltpu.VMEM((TB, BC), jnp.float32),
            pltpu.VMEM((TB, BC), jnp.float32),
            pltpu.VMEM((B, TB, V), jnp.float32),
            pltpu.VMEM((B, TB, V), jnp.float32),
            pltpu.VMEM((B, TB, ROW), jnp.float32),
            pltpu.VMEM((B, TB, ROW), jnp.float32),
            pltpu.VMEM((BC,), jnp.int32),
            pltpu.VMEM((V,), jnp.int32),
            pltpu.SemaphoreType.DMA,
            pltpu.SemaphoreType.DMA,
            pltpu.SemaphoreType.DMA,
            pltpu.SemaphoreType.DMA,
        ],
    )
    return run(inp, variable_encodings, static_channels)


# speedup vs baseline: 2.3555x; 2.3555x over previous
import jax
import jax.numpy as jnp
from jax import lax
from jax.experimental import pallas as pl
from jax.experimental.pallas import tpu as pltpu
from jax.experimental.pallas import tpu_sc as plsc

B = 4
T = 8192
V = 128
NSC = 7
NEC = 8
EXP = 1 + NSC + NEC
ROW = V * EXP
ST = V * NSC
EN = V * NEC
BC = ST + EN
L = 16
NW = 32
TPW = T // NW
TB = 4
NCH = TPW // TB


def _tec_body(inp_hbm, enc_hbm, st_hbm, out_hbm,
              bc0, bc1, in0, in1, out0, out1, idx_v, idxi_v,
              sin0, sin1, sout0, sout1):
    wid = lax.axis_index("s") * 2 + lax.axis_index("c")
    lanes = lax.iota(jnp.int32, 16)
    bc = (bc0, bc1)
    inv = (in0, in1)
    outv = (out0, out1)
    sin = (sin0, sin1)
    sout = (sout0, sout1)

    @pl.loop(0, ST // L)
    def _(r):
        s = r * L + lanes
        v = (s * 9363) >> 16
        idx_v[pl.ds(r * L, L)] = v * 16 + (s - v * 7) + 1

    @pl.loop(0, EN // L)
    def _(r):
        e = r * L + lanes
        idx_v[pl.ds(ST + r * L, L)] = ((e >> 3) << 4) + 8 + (e & 7)

    @pl.loop(0, V // L)
    def _(r):
        v = r * L + lanes
        idxi_v[pl.ds(r * L, L)] = v << 4

    def start_in(k, t):
        pltpu.async_copy(st_hbm.at[pl.ds(t, TB)], bc[k].at[:, pl.ds(0, ST)], sin[k])
        pltpu.async_copy(enc_hbm.at[pl.ds(t, TB)], bc[k].at[:, pl.ds(ST, EN)], sin[k])
        pltpu.async_copy(inp_hbm.at[:, pl.ds(t, TB)], inv[k], sin[k])

    def wait_in(k):
        pltpu.make_async_copy(st_hbm.at[pl.ds(0, TB)], bc[k].at[:, pl.ds(0, ST)], sin[k]).wait()
        pltpu.make_async_copy(enc_hbm.at[pl.ds(0, TB)], bc[k].at[:, pl.ds(ST, EN)], sin[k]).wait()
        pltpu.make_async_copy(inp_hbm.at[:, pl.ds(0, TB)], inv[k], sin[k]).wait()

    def start_out(k, t):
        pltpu.async_copy(outv[k], out_hbm.at[:, pl.ds(t, TB)], sout[k])

    def wait_out(k):
        pltpu.make_async_copy(outv[k], out_hbm.at[:, pl.ds(0, TB)], sout[k]).wait()

    def build(k):
        for j in range(TB):
            rj = jnp.full((L,), j, jnp.int32)

            @plsc.parallel_loop(0, BC // L, unroll=4)
            def _(r, rj=rj, j=j):
                x = bc[k][j, pl.ds(r * L, L)]
                d = idx_v[pl.ds(r * L, L)]
                for b in range(B):
                    rb = jnp.full((L,), b, jnp.int32)
                    plsc.store_scatter(outv[k], [rb, rj, d], x)

            @plsc.parallel_loop(0, V // L, unroll=4)
            def _(r, rj=rj, j=j):
                d = idxi_v[pl.ds(r * L, L)]
                for b in range(B):
                    x = inv[k][b, j, pl.ds(r * L, L)]
                    rb = jnp.full((L,), b, jnp.int32)
                    plsc.store_scatter(outv[k], [rb, rj, d], x)

    t0 = wid * TPW
    start_in(0, t0)
    start_in(1, t0 + TB)
    for k in range(2):
        wait_in(k)
        build(k)
        start_out(k, t0 + k * TB)
        start_in(k, t0 + (k + 2) * TB)

    @pl.loop(0, (NCH - 4) // 2)
    def _(ii):
        c = 2 + ii * 2
        for k in range(2):
            t = t0 + (c + k) * TB
            wait_in(k)
            wait_out(k)
            build(k)
            start_out(k, t)
            start_in(k, t + 2 * TB)

    for k in range(2):
        wait_in(k)
        wait_out(k)
        build(k)
        start_out(k, t0 + (NCH - 2 + k) * TB)
    for k in range(2):
        wait_out(k)


@jax.jit
def kernel(inp, variable_encodings, static_channels):
    run = pl.kernel(
        _tec_body,
        out_type=jax.ShapeDtypeStruct((B, T, ROW), jnp.float32),
        mesh=plsc.VectorSubcoreMesh(core_axis_name="c", subcore_axis_name="s"),
        compiler_params=pltpu.CompilerParams(
            needs_layout_passes=False, disable_bounds_checks=True),
        scratch_types=[
            pltpu.VMEM((TB, BC), jnp.float32),
            pltpu.VMEM((TB, BC), jnp.float32),
            pltpu.VMEM((B, TB, V), jnp.float32),
            pltpu.VMEM((B, TB, V), jnp.float32),
            pltpu.VMEM((B, TB, ROW), jnp.float32),
            pltpu.VMEM((B, TB, ROW), jnp.float32),
            pltpu.VMEM((BC,), jnp.int32),
            pltpu.VMEM((V,), jnp.int32),
            pltpu.SemaphoreType.DMA,
            pltpu.SemaphoreType.DMA,
            pltpu.SemaphoreType.DMA,
            pltpu.SemaphoreType.DMA,
        ],
    )
    return run(inp, variable_encodings, static_channels)
